# baseline (speedup 1.0000x reference)
import functools

import jax
import jax.numpy as jnp
from jax import lax
from jax.experimental import pallas as pl
from jax.experimental.pallas import tpu as pltpu
from jax.experimental.pallas import tpu_sc as plsc

N_NODES = 10000
N_EDGES = 320000
DIM = 128

PAD_NODES = 10240
NC, NS = 2, 16
PAD_EDGES = 327680
EPC = PAD_EDGES // NC
ECHUNK = 2048
N_ECHUNKS = EPC // ECHUNK
UNROLL = 4
N_SG = ECHUNK // (16 * UNROLL)
FIRE = 128
BACKLOG = FIRE + 16 * UNROLL
OWN_ROWS = PAD_NODES // NS


def _mm_body(h_ref, w_ref, n_ref, o_ref):
    o_ref[...] = (
        jnp.dot(h_ref[...], w_ref[...], preferred_element_type=jnp.float32)
        * n_ref[...]
    )


def _matmul_norm(h_pad, weight, norm_pad):
    return pl.pallas_call(
        _mm_body,
        grid=(8,),
        in_specs=[
            pl.BlockSpec((PAD_NODES // 8, DIM), lambda i: (i, 0)),
            pl.BlockSpec((DIM, DIM), lambda i: (0, 0)),
            pl.BlockSpec((PAD_NODES // 8, 1), lambda i: (i, 0)),
        ],
        out_specs=pl.BlockSpec((PAD_NODES // 8, DIM), lambda i: (i, 0)),
        out_shape=jax.ShapeDtypeStruct((PAD_NODES, DIM), jnp.float32),
    )(h_pad, weight, norm_pad)


def _sc_scatter(hw_pad, src, dst):
    mesh = plsc.VectorSubcoreMesh(core_axis_name="c", subcore_axis_name="s")

    @functools.partial(
        pl.kernel,
        mesh=mesh,
        out_type=jax.ShapeDtypeStruct((NC, PAD_NODES, DIM), jnp.float32),
        scratch_types=[
            pltpu.VMEM((ECHUNK,), jnp.int32),
            pltpu.VMEM((ECHUNK,), jnp.int32),
            pltpu.VMEM((BACKLOG,), jnp.int32),
            pltpu.VMEM((BACKLOG,), jnp.int32),
            pltpu.VMEM((2, FIRE), jnp.int32),
            pltpu.VMEM((2, FIRE), jnp.int32),
            pltpu.VMEM((2, FIRE, DIM), jnp.float32),
            pltpu.VMEM((16, DIM), jnp.float32),
            pltpu.VMEM_SHARED((PAD_NODES, DIM), jnp.float32),
            pltpu.SemaphoreType.DMA,
        ],
        compiler_params=pltpu.CompilerParams(needs_layout_passes=False),
    )
    def k(hw_hbm, src_hbm, dst_hbm, out_hbm,
          srcchunk_v, dstchunk_v, fsrc_buf, fdst_buf, fsrc_fire, fdst_fire,
          rows_v, z_v, acc_sh, sem):
        cid = lax.axis_index("c")
        sid = lax.axis_index("s")
        lo = sid * OWN_ROWS
        hi = lo + OWN_ROWS

        zeros16 = jnp.zeros((16,), jnp.float32)

        def zero_body(i, _):
            z_v[i // 8, pl.ds((i % 8) * 16, 16)] = zeros16
            return 0

        lax.fori_loop(0, 16 * (DIM // 16), zero_body, 0)

        def zslab_body(i, _):
            pltpu.sync_copy(z_v, acc_sh.at[pl.ds(lo + i * 16, 16)])
            return 0

        lax.fori_loop(0, OWN_ROWS // 16, zslab_body, 0)

        def drain_and_scatter(par):
            pltpu.make_async_copy(
                hw_hbm.at[pl.ds(0, FIRE)], rows_v.at[par], sem).wait()

        def issue_gather(par):
            pltpu.async_copy(hw_hbm.at[fsrc_fire.at[par]], rows_v.at[par], sem)

        ebase = cid * EPC

        def sg_body(t, carry):
            cnt, parity, pending = carry
            base = t * (16 * UNROLL)
            offs = cnt
            for u in range(UNROLL):
                s16 = srcchunk_v[pl.ds(base + u * 16, 16)]
                d16 = dstchunk_v[pl.ds(base + u * 16, 16)]
                m = (d16 >= lo) & (d16 < hi)
                pc = plsc.all_reduce_population_count(m)
                plsc.store_compressed(fsrc_buf.at[pl.ds(offs, 16)], s16,
                                      mask=m)
                plsc.store_compressed(fdst_buf.at[pl.ds(offs, 16)], d16,
                                      mask=m)
                offs = offs + pc[0]
            cnt = offs
            fired = cnt >= FIRE

            @pl.when(fired)
            def _():
                @pl.when(pending == 1)
                def _():
                    drain_and_scatter(1 - parity)

                for j in range(FIRE // 16):
                    fsrc_fire[parity, pl.ds(j * 16, 16)] = (
                        fsrc_buf[pl.ds(j * 16, 16)])
                    fdst_fire[parity, pl.ds(j * 16, 16)] = (
                        fdst_buf[pl.ds(j * 16, 16)])
                issue_gather(parity)
                for j in range(UNROLL):
                    fsrc_buf[pl.ds(j * 16, 16)] = (
                        fsrc_buf[pl.ds(FIRE + j * 16, 16)])
                    fdst_buf[pl.ds(j * 16, 16)] = (
                        fdst_buf[pl.ds(FIRE + j * 16, 16)])

            cnt = jnp.where(fired, cnt - FIRE, cnt)
            parity = jnp.where(fired, 1 - parity, parity)
            pending = jnp.where(fired, 1, pending)
            return (cnt, parity, pending)

        def chunk_body(ec, carry):
            off = ebase + ec * ECHUNK
            pltpu.sync_copy(src_hbm.at[pl.ds(off, ECHUNK)], srcchunk_v)
            pltpu.sync_copy(dst_hbm.at[pl.ds(off, ECHUNK)], dstchunk_v)
            return lax.fori_loop(0, N_SG, sg_body, carry)

        cnt, parity, pending = lax.fori_loop(
            0, N_ECHUNKS, chunk_body, (0, 0, 0))

        @pl.when(pending == 1)
        def _():
            drain_and_scatter(1 - parity)

        for j in range(FIRE // 16):
            pos = lax.iota(jnp.int32, 16) + j * 16
            m = pos < cnt
            fsrc_fire[parity, pl.ds(j * 16, 16)] = jnp.where(
                m, fsrc_buf[pl.ds(j * 16, 16)], N_NODES)
            fdst_fire[parity, pl.ds(j * 16, 16)] = jnp.where(
                m, fdst_buf[pl.ds(j * 16, 16)], lo)
        issue_gather(parity)
        drain_and_scatter(parity)

        pltpu.sync_copy(
            acc_sh.at[pl.ds(lo, OWN_ROWS)],
            out_hbm.at[cid, pl.ds(lo, OWN_ROWS)],
        )

    return k(hw_pad, src, dst)


def _comb_body(p0_ref, p1_ref, n_ref, b_ref, o_ref):
    o_ref[...] = (p0_ref[0] + p1_ref[0]) * n_ref[...] + b_ref[...]


def _combine(partials, norm, bias2d):
    return pl.pallas_call(
        _comb_body,
        grid=(10,),
        in_specs=[
            pl.BlockSpec((1, 1000, DIM), lambda i: (0, i, 0)),
            pl.BlockSpec((1, 1000, DIM), lambda i: (1, i, 0)),
            pl.BlockSpec((1000, 1), lambda i: (i, 0)),
            pl.BlockSpec((1, DIM), lambda i: (0, 0)),
        ],
        out_specs=pl.BlockSpec((1000, DIM), lambda i: (i, 0)),
        out_shape=jax.ShapeDtypeStruct((N_NODES, DIM), jnp.float32),
    )(partials, partials, norm, bias2d)


def kernel(h, norm, edge_index, weight, bias):
    h_pad = jnp.pad(h, ((0, PAD_NODES - N_NODES), (0, 0)))
    norm_pad = jnp.pad(norm, ((0, PAD_NODES - N_NODES), (0, 0)))
    npad = PAD_EDGES - N_EDGES
    src = jnp.concatenate(
        [edge_index[0].astype(jnp.int32), jnp.full((npad,), N_NODES, jnp.int32)]
    )
    dst = jnp.concatenate(
        [edge_index[1].astype(jnp.int32), jnp.full((npad,), N_NODES, jnp.int32)]
    )
    hw_pad = _matmul_norm(h_pad, weight, norm_pad)
    partials = _sc_scatter(hw_pad, src, dst)
    return _combine(partials, norm, jnp.reshape(bias, (1, DIM)))


# speedup vs baseline: 1.0842x; 1.0842x over previous
import functools

import jax
import jax.numpy as jnp
from jax import lax
from jax.experimental import pallas as pl
from jax.experimental.pallas import tpu as pltpu
from jax.experimental.pallas import tpu_sc as plsc

N_NODES = 10000
N_EDGES = 320000
DIM = 128

PAD_NODES = 10240
NC, NS = 2, 16
PAD_EDGES = 327680
ECHUNK = 2048
N_ECHUNKS = PAD_EDGES // ECHUNK
UNROLL = 4
N_SG = ECHUNK // (16 * UNROLL)
FIRE = 128
BACKLOG = FIRE + 16 * UNROLL
NB = 4
OWN_ROWS = PAD_NODES // (NC * NS)
CORE_ROWS = PAD_NODES // NC


def _mm_body(h_ref, w_ref, n_ref, o_ref):
    o_ref[...] = (
        jnp.dot(h_ref[...], w_ref[...], preferred_element_type=jnp.float32)
        * n_ref[...]
    )


def _matmul_norm(h_pad, weight, norm_pad):
    return pl.pallas_call(
        _mm_body,
        grid=(8,),
        in_specs=[
            pl.BlockSpec((PAD_NODES // 8, DIM), lambda i: (i, 0)),
            pl.BlockSpec((DIM, DIM), lambda i: (0, 0)),
            pl.BlockSpec((PAD_NODES // 8, 1), lambda i: (i, 0)),
        ],
        out_specs=pl.BlockSpec((PAD_NODES // 8, DIM), lambda i: (i, 0)),
        out_shape=jax.ShapeDtypeStruct((PAD_NODES, DIM), jnp.float32),
    )(h_pad, weight, norm_pad)


def _sc_scatter(hw_pad, src, dst):
    mesh = plsc.VectorSubcoreMesh(core_axis_name="c", subcore_axis_name="s")

    @functools.partial(
        pl.kernel,
        mesh=mesh,
        out_type=jax.ShapeDtypeStruct((PAD_NODES, DIM), jnp.float32),
        scratch_types=[
            pltpu.VMEM((2, ECHUNK), jnp.int32),
            pltpu.VMEM((2, ECHUNK), jnp.int32),
            pltpu.VMEM((BACKLOG,), jnp.int32),
            pltpu.VMEM((BACKLOG,), jnp.int32),
            pltpu.VMEM((NB, FIRE), jnp.int32),
            pltpu.VMEM((NB, FIRE), jnp.int32),
            pltpu.VMEM((NB, FIRE, DIM), jnp.float32),
            pltpu.VMEM((16, DIM), jnp.float32),
            pltpu.VMEM_SHARED((CORE_ROWS, DIM), jnp.float32),
            pltpu.SemaphoreType.DMA,
            pltpu.SemaphoreType.DMA,
        ],
        compiler_params=pltpu.CompilerParams(needs_layout_passes=False),
    )
    def k(hw_hbm, src_hbm, dst_hbm, out_hbm,
          srcchunk_v, dstchunk_v, fsrc_buf, fdst_buf, fsrc_fire, fdst_fire,
          rows_v, z_v, acc_sh, sem_g, sem_c):
        cid = lax.axis_index("c")
        sid = lax.axis_index("s")
        core_base = cid * CORE_ROWS
        llo = sid * OWN_ROWS
        lo = core_base + llo
        hi = lo + OWN_ROWS

        def issue_chunk(ec, pb):
            off = ec * ECHUNK
            pltpu.async_copy(
                src_hbm.at[pl.ds(off, ECHUNK)], srcchunk_v.at[pb], sem_c)
            pltpu.async_copy(
                dst_hbm.at[pl.ds(off, ECHUNK)], dstchunk_v.at[pb], sem_c)

        def wait_chunk(pb):
            pltpu.make_async_copy(
                src_hbm.at[pl.ds(0, ECHUNK)], srcchunk_v.at[pb], sem_c).wait()
            pltpu.make_async_copy(
                dst_hbm.at[pl.ds(0, ECHUNK)], dstchunk_v.at[pb], sem_c).wait()

        issue_chunk(0, 0)

        zeros16 = jnp.zeros((16,), jnp.float32)

        def zero_body(i, _):
            z_v[i // 8, pl.ds((i % 8) * 16, 16)] = zeros16
            return 0

        lax.fori_loop(0, 16 * (DIM // 16), zero_body, 0)

        def zslab_body(i, _):
            pltpu.sync_copy(z_v, acc_sh.at[pl.ds(llo + i * 16, 16)])
            return 0

        lax.fori_loop(0, OWN_ROWS // 16, zslab_body, 0)

        def drain_and_scatter(q):
            pltpu.make_async_copy(
                hw_hbm.at[pl.ds(0, FIRE)], rows_v.at[q], sem_g).wait()
            pltpu.sync_copy(
                rows_v.at[q], acc_sh.at[fdst_fire.at[q]], add=True)

        def issue_gather(q):
            pltpu.async_copy(hw_hbm.at[fsrc_fire.at[q]], rows_v.at[q], sem_g)

        def sg_body(t, carry):
            cnt, nf, pb = carry
            base = t * (16 * UNROLL)
            offs = cnt
            for u in range(UNROLL):
                s16 = srcchunk_v[pb, pl.ds(base + u * 16, 16)]
                d16 = dstchunk_v[pb, pl.ds(base + u * 16, 16)]
                m = (d16 >= lo) & (d16 < hi)
                pc = plsc.all_reduce_population_count(m)
                plsc.store_compressed(fsrc_buf.at[pl.ds(offs, 16)], s16,
                                      mask=m)
                plsc.store_compressed(fdst_buf.at[pl.ds(offs, 16)],
                                      d16 - core_base, mask=m)
                offs = offs + pc[0]
            cnt = offs
            fired = cnt >= FIRE

            @pl.when(fired)
            def _():
                q = nf % NB

                @pl.when(nf >= NB - 1)
                def _():
                    drain_and_scatter((nf - (NB - 1)) % NB)

                for j in range(FIRE // 16):
                    fsrc_fire[q, pl.ds(j * 16, 16)] = (
                        fsrc_buf[pl.ds(j * 16, 16)])
                    fdst_fire[q, pl.ds(j * 16, 16)] = (
                        fdst_buf[pl.ds(j * 16, 16)])
                issue_gather(q)
                for j in range(UNROLL):
                    fsrc_buf[pl.ds(j * 16, 16)] = (
                        fsrc_buf[pl.ds(FIRE + j * 16, 16)])
                    fdst_buf[pl.ds(j * 16, 16)] = (
                        fdst_buf[pl.ds(FIRE + j * 16, 16)])

            cnt = jnp.where(fired, cnt - FIRE, cnt)
            nf = jnp.where(fired, nf + 1, nf)
            return (cnt, nf, pb)

        def chunk_body(ec, carry):
            cnt, nf, pb = carry
            wait_chunk(pb)

            @pl.when(ec + 1 < N_ECHUNKS)
            def _():
                issue_chunk(ec + 1, 1 - pb)

            cnt, nf, _ = lax.fori_loop(0, N_SG, sg_body, (cnt, nf, pb))
            return (cnt, nf, 1 - pb)

        cnt, nf, _ = lax.fori_loop(0, N_ECHUNKS, chunk_body, (0, 0, 0))

        for i in range(NB - 1):
            b = nf - (NB - 1) + i

            @pl.when((b >= 0) & (b < nf))
            def _():
                drain_and_scatter(b % NB)

        q = nf % NB
        for j in range(FIRE // 16):
            pos = lax.iota(jnp.int32, 16) + j * 16
            m = pos < cnt
            fsrc_fire[q, pl.ds(j * 16, 16)] = jnp.where(
                m, fsrc_buf[pl.ds(j * 16, 16)], N_NODES)
            fdst_fire[q, pl.ds(j * 16, 16)] = jnp.where(
                m, fdst_buf[pl.ds(j * 16, 16)], llo)
        issue_gather(q)
        drain_and_scatter(q)

        pltpu.sync_copy(
            acc_sh.at[pl.ds(llo, OWN_ROWS)],
            out_hbm.at[pl.ds(lo, OWN_ROWS)],
        )

    return k(hw_pad, src, dst)


def _comb_body(p_ref, n_ref, b_ref, o_ref):
    o_ref[...] = p_ref[...] * n_ref[...] + b_ref[...]


def _combine(agg, norm, bias2d):
    return pl.pallas_call(
        _comb_body,
        grid=(10,),
        in_specs=[
            pl.BlockSpec((1000, DIM), lambda i: (i, 0)),
            pl.BlockSpec((1000, 1), lambda i: (i, 0)),
            pl.BlockSpec((1, DIM), lambda i: (0, 0)),
        ],
        out_specs=pl.BlockSpec((1000, DIM), lambda i: (i, 0)),
        out_shape=jax.ShapeDtypeStruct((N_NODES, DIM), jnp.float32),
    )(agg, norm, bias2d)


def kernel(h, norm, edge_index, weight, bias):
    h_pad = jnp.pad(h, ((0, PAD_NODES - N_NODES), (0, 0)))
    norm_pad = jnp.pad(norm, ((0, PAD_NODES - N_NODES), (0, 0)))
    npad = PAD_EDGES - N_EDGES
    src = jnp.concatenate(
        [edge_index[0].astype(jnp.int32), jnp.full((npad,), N_NODES, jnp.int32)]
    )
    dst = jnp.concatenate(
        [edge_index[1].astype(jnp.int32), jnp.full((npad,), N_NODES, jnp.int32)]
    )
    hw_pad = _matmul_norm(h_pad, weight, norm_pad)
    agg = _sc_scatter(hw_pad, src, dst)
    return _combine(agg, norm, jnp.reshape(bias, (1, DIM)))
